# baseline (speedup 1.0000x reference)
import functools

import jax
import jax.numpy as jnp
from jax import lax
from jax.experimental import pallas as pl
from jax.experimental.pallas import tpu as pltpu
from jax.experimental.pallas import tpu_sc as plsc

N = 10000
D = 128
NC = 2
NS = 16
NT = NC * NS
CH = 128
CPT = 80
EPT = CPT * CH
EP = NT * EPT
NPAD = 10080
RPS = 632

_mesh = plsc.VectorSubcoreMesh(core_axis_name="c", subcore_axis_name="s")
_sc_params = pltpu.CompilerParams(needs_layout_passes=False)
_sc_params_nt = pltpu.CompilerParams(
    needs_layout_passes=False, use_tc_tiling_on_sc=False
)


@functools.partial(
    pl.kernel,
    mesh=_mesh,
    out_type=jax.ShapeDtypeStruct((NT, NPAD), jnp.float32),
    compiler_params=_sc_params,
    scratch_types=[
        pltpu.VMEM((EPT,), jnp.int32),
        pltpu.VMEM((NPAD,), jnp.float32),
    ],
)
def _deg_kernel(dst_hbm, out_hbm, dst_v, hist_v):
    c = lax.axis_index("c")
    s = lax.axis_index("s")
    t = c * NS + s
    pltpu.sync_copy(dst_hbm.at[t], dst_v)

    def zero_body(i, carry):
        hist_v[pl.ds(i * 16, 16)] = jnp.zeros((16,), jnp.float32)
        return carry

    lax.fori_loop(0, NPAD // 16, zero_body, 0)

    ones = jnp.ones((16,), jnp.float32)

    def body(i, carry):
        idx = dst_v[pl.ds(i * 16, 16)]
        plsc.addupdate_scatter(hist_v, [idx], ones)
        return carry

    lax.fori_loop(0, EPT // 16, body, 0)
    pltpu.sync_copy(hist_v, out_hbm.at[t])


@functools.partial(
    pl.kernel,
    mesh=_mesh,
    out_type=jax.ShapeDtypeStruct((NC, NPAD, D), jnp.float32),
    compiler_params=_sc_params_nt,
    scratch_types=[
        pltpu.VMEM((CH, D // 2), jnp.int32),
        pltpu.VMEM((CH, D // 2), jnp.int32),
        pltpu.VMEM((CH, D), jnp.float32),
        pltpu.VMEM((CH, D), jnp.float32),
        pltpu.VMEM((2, CH), jnp.int32),
        pltpu.VMEM((2, CH), jnp.int32),
        pltpu.VMEM((2, CH), jnp.int32),
        pltpu.VMEM((2, CH), jnp.int32),
        pltpu.VMEM_SHARED((NPAD, D), jnp.float32),
        pltpu.SemaphoreType.DMA,
        pltpu.SemaphoreType.DMA,
        pltpu.SemaphoreType.DMA,
        pltpu.SemaphoreType.DMA,
        pltpu.SemaphoreType.DMA,
        pltpu.SemaphoreType.DMA,
        pltpu.SemaphoreType.DMA,
        pltpu.SemaphoreType.DMA,
    ],
)
def _edge_kernel(
    h2_hbm, eidx_hbm, out_hbm,
    b0, b1, f0, f1, i0, i1, i2, i3, acc,
    gs0, gs1, ss0, ss1, is0, is1, is2, is3,
):
    c = lax.axis_index("c")
    s = lax.axis_index("s")
    t = c * NS + s
    B = [b0, b1]
    F = [f0, f1]
    I = [i0, i1, i2, i3]
    gsem = [gs0, gs1]
    ssem = [ss0, ss1]
    isem = [is0, is1, is2, is3]

    def istart(g, sl):
        pltpu.async_copy(eidx_hbm.at[t, g], I[sl], isem[sl])

    def iwait(g, sl):
        pltpu.make_async_copy(eidx_hbm.at[t, g], I[sl], isem[sl]).wait()

    def gstart(sl2, sl4):
        pltpu.async_copy(h2_hbm.at[I[sl4].at[0]], B[sl2], gsem[sl2])

    def gwait(sl2, sl4):
        pltpu.make_async_copy(h2_hbm.at[I[sl4].at[0]], B[sl2], gsem[sl2]).wait()

    def sstart(sl2, sl4):
        pltpu.async_copy(F[sl2], acc.at[I[sl4].at[1]], ssem[sl2], add=True)

    def swait(sl2, sl4):
        pltpu.make_async_copy(F[sl2], acc.at[I[sl4].at[1]], ssem[sl2]).wait()

    def convert(sl2):
        def crow(r, carry):
            for q in range(4):
                v = plsc.bitcast(B[sl2][r, pl.ds(q * 16, 16)], jnp.bfloat16)
                lo, hi = plsc.unpack(v, format=plsc.PackFormat.INTERLEAVED)
                F[sl2][r, pl.ds(q * 16, 16)] = lo
                F[sl2][r, pl.ds(64 + q * 16, 16)] = hi
            return carry

        lax.fori_loop(0, CH, crow, 0)

    istart(0, 0)
    istart(1, 1)
    istart(2, 2)
    iwait(0, 0)
    gstart(0, 0)
    iwait(1, 1)
    gstart(1, 1)

    def zbody(i, carry):
        r = i // 8
        j = i % 8
        f0[r, pl.ds(j * 16, 16)] = jnp.zeros((16,), jnp.float32)
        return carry

    lax.fori_loop(0, CH * 8, zbody, 0)

    def zcopy(k, carry):
        pltpu.sync_copy(f0, acc.at[pl.ds(s * RPS + k * CH, CH)])
        return carry

    lax.fori_loop(0, 4, zcopy, 0)

    @pl.when(s < NS - 1)
    def _():
        pltpu.sync_copy(
            f0.at[pl.ds(0, RPS - 4 * CH)],
            acc.at[pl.ds(s * RPS + 4 * CH, RPS - 4 * CH)],
        )

    @pl.when(s == NS - 1)
    def _():
        pltpu.sync_copy(
            f0.at[pl.ds(0, NPAD - 15 * RPS - 4 * CH)],
            acc.at[pl.ds(s * RPS + 4 * CH, NPAD - 15 * RPS - 4 * CH)],
        )

    plsc.subcore_barrier()

    nk = CPT // 4

    def step(k, carry):
        for b in range(4):
            g = 4 * k + b
            sl2 = b % 2
            gwait(sl2, b)
            convert(sl2)
            sstart(sl2, b)
            if b == 0:
                @pl.when(k > 0)
                def _():
                    swait(1, 3)
                istart(g + 3, 3)
                iwait(g + 2, 2)
                gstart(sl2, 2)
            else:
                swait((b - 1) % 2, b - 1)
                if b == 1:
                    @pl.when(k < nk - 1)
                    def _():
                        istart(g + 3, 0)
                    iwait(g + 2, 3)
                    gstart(sl2, 3)
                else:
                    @pl.when(k < nk - 1)
                    def _():
                        istart(g + 3, (b + 3) % 4)
                        iwait(g + 2, (b + 2) % 4)
                        gstart(sl2, (b + 2) % 4)
            return_val = carry
        return return_val

    lax.fori_loop(0, nk, step, 0)
    swait(1, 3)
    plsc.subcore_barrier()

    def ocopy(k, carry):
        pltpu.sync_copy(
            acc.at[pl.ds(s * RPS + k * CH, CH)],
            out_hbm.at[c, pl.ds(s * RPS + k * CH, CH)],
        )
        return carry

    lax.fori_loop(0, 4, ocopy, 0)

    @pl.when(s < NS - 1)
    def _():
        pltpu.sync_copy(
            acc.at[pl.ds(s * RPS + 4 * CH, RPS - 4 * CH)],
            out_hbm.at[c, pl.ds(s * RPS + 4 * CH, RPS - 4 * CH)],
        )

    @pl.when(s == NS - 1)
    def _():
        pltpu.sync_copy(
            acc.at[pl.ds(s * RPS + 4 * CH, NPAD - 15 * RPS - 4 * CH)],
            out_hbm.at[c, pl.ds(s * RPS + 4 * CH, NPAD - 15 * RPS - 4 * CH)],
        )


def _m0_body(x_ref, w_ref, wp_ref, m_ref, m2_ref):
    xv = x_ref[...]
    m_ref[...] = jnp.dot(xv, w_ref[...], preferred_element_type=jnp.float32)
    m2_ref[...] = jnp.dot(xv, wp_ref[...], preferred_element_type=jnp.float32)


def _m0(x, w, wp):
    return pl.pallas_call(
        _m0_body,
        out_shape=(
            jax.ShapeDtypeStruct((N, D), jnp.float32),
            jax.ShapeDtypeStruct((N, D), jnp.float32),
        ),
    )(x, w, wp)


def _scale_body(hists_ref, m_ref, m2_ref, h_ref, h2_ref, dis_ref):
    deg = jnp.sum(hists_ref[:, :N], axis=0) + 1.0
    dis = lax.rsqrt(deg)[:, None]
    dis_ref[...] = dis
    h_ref[...] = m_ref[...] * dis
    h2_ref[...] = (m2_ref[...] * dis).astype(jnp.bfloat16)


def _scale(hists, m, m2):
    return pl.pallas_call(
        _scale_body,
        out_shape=(
            jax.ShapeDtypeStruct((N, D), jnp.float32),
            jax.ShapeDtypeStruct((N, D), jnp.bfloat16),
            jax.ShapeDtypeStruct((N, 1), jnp.float32),
        ),
    )(hists, m, m2)


def _fuse_body(p_ref, h_ref, dis_ref, b_ref, w_ref, wp_ref, o_ref, o2_ref):
    dis = dis_ref[...]
    a = p_ref[0, :N] + p_ref[1, :N] + h_ref[...]
    x = jnp.maximum(a * dis + b_ref[...], 0.0)
    u = x * dis
    o_ref[...] = jnp.dot(u, w_ref[...], preferred_element_type=jnp.float32)
    o2_ref[...] = jnp.dot(
        u, wp_ref[...], preferred_element_type=jnp.float32
    ).astype(jnp.bfloat16)


def _fuse(p, h, dis, b, w, wp):
    return pl.pallas_call(
        _fuse_body,
        out_shape=(
            jax.ShapeDtypeStruct((N, D), jnp.float32),
            jax.ShapeDtypeStruct((N, D), jnp.bfloat16),
        ),
    )(p, h, dis, b, w, wp)


def _final_body(p_ref, h_ref, dis_ref, b_ref, o_ref):
    a = p_ref[0, :N] + p_ref[1, :N] + h_ref[...]
    o_ref[...] = a * dis_ref[...] + b_ref[...]


def _final(p, h, dis, b):
    return pl.pallas_call(
        _final_body,
        out_shape=jax.ShapeDtypeStruct((N, D), jnp.float32),
    )(p, h, dis, b)


def kernel(x, edge_index, W0, b0, W1, b1, W2, b2, W3, b3):
    src = edge_index[0].astype(jnp.int32)
    dst = edge_index[1].astype(jnp.int32)
    e = src.shape[0]
    pad_n = EP - e
    ar = jnp.arange(pad_n, dtype=jnp.int32)
    src_p = jnp.concatenate([src, ar % N]).reshape(NT, CPT, CH)
    dst_p = jnp.concatenate([dst, N + ar % (NPAD - N)]).reshape(NT, CPT, CH)
    eidx = jnp.stack([src_p, dst_p], axis=2)
    dst_flat = dst_p.reshape(NT, EPT)

    perm = jnp.stack(
        [jnp.arange(64, dtype=jnp.int32), jnp.arange(64, dtype=jnp.int32) + 64],
        axis=1,
    ).reshape(D)
    Ws = [W0, W1, W2, W3]
    Wps = [w[:, perm] for w in Ws]

    def to_i32(hb):
        return jax.lax.bitcast_convert_type(
            hb.reshape(N, D // 2, 2), jnp.int32
        )

    m, m2 = _m0(x, W0, Wps[0])
    hists = _deg_kernel(dst_flat)
    h, h2, dis = _scale(hists, m, m2)
    b_prev = [b0, b1, b2]
    for i in range(3):
        p = _edge_kernel(to_i32(h2), eidx)
        h, h2 = _fuse(p, h, dis, b_prev[i].reshape(1, D), Ws[i + 1], Wps[i + 1])
    p = _edge_kernel(to_i32(h2), eidx)
    return _final(p, h, dis, b3.reshape(1, D))


# speedup vs baseline: 1.9599x; 1.9599x over previous
import functools

import jax
import jax.numpy as jnp
from jax import lax
from jax.experimental import pallas as pl
from jax.experimental.pallas import tpu as pltpu
from jax.experimental.pallas import tpu_sc as plsc

N = 10000
D = 128
NC = 2
NS = 16
NT = NC * NS
CH = 128
CPT = 80
EPT = CPT * CH
EP = NT * EPT
NPAD = 10240
RPS = NPAD // NS

_mesh = plsc.VectorSubcoreMesh(core_axis_name="c", subcore_axis_name="s")
_sc_params = pltpu.CompilerParams(needs_layout_passes=False)


@functools.partial(
    pl.kernel,
    mesh=_mesh,
    out_type=jax.ShapeDtypeStruct((NT, NPAD), jnp.float32),
    compiler_params=_sc_params,
    scratch_types=[
        pltpu.VMEM((EPT,), jnp.int32),
        pltpu.VMEM((NPAD,), jnp.float32),
    ],
)
def _deg_kernel(dst_hbm, out_hbm, dst_v, hist_v):
    c = lax.axis_index("c")
    s = lax.axis_index("s")
    t = c * NS + s
    pltpu.sync_copy(dst_hbm.at[t], dst_v)

    def zero_body(i, carry):
        hist_v[pl.ds(i * 16, 16)] = jnp.zeros((16,), jnp.float32)
        return carry

    lax.fori_loop(0, NPAD // 16, zero_body, 0)

    ones = jnp.ones((16,), jnp.float32)

    def body(i, carry):
        idx = dst_v[pl.ds(i * 16, 16)]
        plsc.addupdate_scatter(hist_v, [idx], ones)
        return carry

    lax.fori_loop(0, EPT // 16, body, 0)
    pltpu.sync_copy(hist_v, out_hbm.at[t])


@functools.partial(
    pl.kernel,
    mesh=_mesh,
    out_type=jax.ShapeDtypeStruct((NC, NPAD, D), jnp.float32),
    compiler_params=_sc_params,
    scratch_types=[
        pltpu.VMEM((CPT // 2, CH), jnp.int32),
        pltpu.VMEM((CPT // 2, CH), jnp.int32),
        pltpu.VMEM((CH, D), jnp.float32),
        pltpu.VMEM((CH, D), jnp.float32),
        pltpu.VMEM_SHARED((NPAD, D), jnp.float32),
        pltpu.SemaphoreType.DMA,
        pltpu.SemaphoreType.DMA,
    ],
)
def _edge_kernel(
    h_hbm, src_hbm, dst_hbm, out_hbm, src_v, dst_v, rows_a, rows_b, acc, sem_a, sem_b
):
    c = lax.axis_index("c")
    s = lax.axis_index("s")
    t = c * NS + s
    hcpt = CPT // 2

    pltpu.sync_copy(src_hbm.at[t, pl.ds(0, hcpt)], src_v)
    pltpu.sync_copy(dst_hbm.at[t, pl.ds(0, hcpt)], dst_v)
    pltpu.async_copy(h_hbm.at[src_v.at[0]], rows_a, sem_a)

    def zbody(i, carry):
        r = i // 8
        j = i % 8
        rows_b[r, pl.ds(j * 16, 16)] = jnp.zeros((16,), jnp.float32)
        return carry

    lax.fori_loop(0, CH * 8, zbody, 0)

    def zcopy(k, carry):
        pltpu.sync_copy(rows_b, acc.at[pl.ds(s * RPS + k * CH, CH)])
        return carry

    lax.fori_loop(0, RPS // CH, zcopy, 0)
    plsc.subcore_barrier()

    npair = hcpt // 2
    for half in range(2):
        if half:
            pltpu.sync_copy(src_hbm.at[t, pl.ds(half * hcpt, hcpt)], src_v)
            pltpu.sync_copy(dst_hbm.at[t, pl.ds(half * hcpt, hcpt)], dst_v)
            pltpu.async_copy(h_hbm.at[src_v.at[0]], rows_a, sem_a)

        def body(k, carry):
            g0 = 2 * k
            g1 = g0 + 1
            pltpu.async_copy(h_hbm.at[src_v.at[g1]], rows_b, sem_b)
            pltpu.make_async_copy(h_hbm.at[src_v.at[g0]], rows_a, sem_a).wait()
            pltpu.sync_copy(rows_a, acc.at[dst_v.at[g0]], add=True)

            @pl.when(k < npair - 1)
            def _():
                pltpu.async_copy(h_hbm.at[src_v.at[g0 + 2]], rows_a, sem_a)

            pltpu.make_async_copy(h_hbm.at[src_v.at[g1]], rows_b, sem_b).wait()
            pltpu.sync_copy(rows_b, acc.at[dst_v.at[g1]], add=True)
            return carry

        lax.fori_loop(0, npair, body, 0)
    plsc.subcore_barrier()

    def ocopy(k, carry):
        pltpu.sync_copy(
            acc.at[pl.ds(s * RPS + k * CH, CH)],
            out_hbm.at[c, pl.ds(s * RPS + k * CH, CH)],
        )
        return carry

    lax.fori_loop(0, RPS // CH, ocopy, 0)


def _m0_body(x_ref, w_ref, m_ref):
    m_ref[...] = jnp.dot(x_ref[...], w_ref[...], preferred_element_type=jnp.float32)


def _m0(x, w):
    return pl.pallas_call(
        _m0_body,
        out_shape=jax.ShapeDtypeStruct((N, D), jnp.float32),
    )(x, w)


def _scale_body(hists_ref, m_ref, h_ref, dis_ref):
    deg = jnp.sum(hists_ref[:, :N], axis=0) + 1.0
    dis = lax.rsqrt(deg)[:, None]
    dis_ref[...] = dis
    h_ref[...] = m_ref[...] * dis


def _scale(hists, m):
    return pl.pallas_call(
        _scale_body,
        out_shape=(
            jax.ShapeDtypeStruct((N, D), jnp.float32),
            jax.ShapeDtypeStruct((N, 1), jnp.float32),
        ),
    )(hists, m)


def _fuse_body(p_ref, h_ref, dis_ref, b_ref, w_ref, o_ref):
    dis = dis_ref[...]
    a = p_ref[0, :N] + p_ref[1, :N] + h_ref[...]
    x = jnp.maximum(a * dis + b_ref[...], 0.0)
    o_ref[...] = jnp.dot(x * dis, w_ref[...], preferred_element_type=jnp.float32)


def _fuse(p, h, dis, b, w):
    return pl.pallas_call(
        _fuse_body,
        out_shape=jax.ShapeDtypeStruct((N, D), jnp.float32),
    )(p, h, dis, b, w)


def _final_body(p_ref, h_ref, dis_ref, b_ref, o_ref):
    a = p_ref[0, :N] + p_ref[1, :N] + h_ref[...]
    o_ref[...] = a * dis_ref[...] + b_ref[...]


def _final(p, h, dis, b):
    return pl.pallas_call(
        _final_body,
        out_shape=jax.ShapeDtypeStruct((N, D), jnp.float32),
    )(p, h, dis, b)


def kernel(x, edge_index, W0, b0, W1, b1, W2, b2, W3, b3):
    src = edge_index[0].astype(jnp.int32)
    dst = edge_index[1].astype(jnp.int32)
    e = src.shape[0]
    pad_n = EP - e
    ar = jnp.arange(pad_n, dtype=jnp.int32)
    src_p = jnp.concatenate([src, ar % N]).reshape(NT, CPT, CH)
    dst_p = jnp.concatenate([dst, N + ar % (NPAD - N)]).reshape(NT, CPT, CH)
    dst_flat = dst_p.reshape(NT, EPT)

    m = _m0(x, W0)
    hists = _deg_kernel(dst_flat)
    h, dis = _scale(hists, m)
    b_prev = [b0, b1, b2]
    w_next = [W1, W2, W3]
    for i in range(3):
        p = _edge_kernel(h, src_p, dst_p)
        h = _fuse(p, h, dis, b_prev[i].reshape(1, D), w_next[i])
    p = _edge_kernel(h, src_p, dst_p)
    return _final(p, h, dis, b3.reshape(1, D))


# speedup vs baseline: 2.0955x; 1.0692x over previous
import functools

import jax
import jax.numpy as jnp
from jax import lax
from jax.experimental import pallas as pl
from jax.experimental.pallas import tpu as pltpu
from jax.experimental.pallas import tpu_sc as plsc

N = 10000
D = 128
NC = 2
NS = 16
NT = NC * NS
CH = 128
CPT = 84
EPT = CPT * CH
EP = NT * EPT
NPAD = 10080
RPS = 632

_mesh = plsc.VectorSubcoreMesh(core_axis_name="c", subcore_axis_name="s")
_sc_params = pltpu.CompilerParams(needs_layout_passes=False)


@functools.partial(
    pl.kernel,
    mesh=_mesh,
    out_type=jax.ShapeDtypeStruct((NT, NPAD), jnp.float32),
    compiler_params=_sc_params,
    scratch_types=[
        pltpu.VMEM((EPT,), jnp.int32),
        pltpu.VMEM((NPAD,), jnp.float32),
    ],
)
def _deg_kernel(dst_hbm, out_hbm, dst_v, hist_v):
    c = lax.axis_index("c")
    s = lax.axis_index("s")
    t = c * NS + s
    pltpu.sync_copy(dst_hbm.at[t], dst_v)

    def zero_body(i, carry):
        hist_v[pl.ds(i * 16, 16)] = jnp.zeros((16,), jnp.float32)
        return carry

    lax.fori_loop(0, NPAD // 16, zero_body, 0)

    ones = jnp.ones((16,), jnp.float32)

    def body(i, carry):
        idx = dst_v[pl.ds(i * 16, 16)]
        plsc.addupdate_scatter(hist_v, [idx], ones)
        return carry

    lax.fori_loop(0, EPT // 16, body, 0)
    pltpu.sync_copy(hist_v, out_hbm.at[t])


@functools.partial(
    pl.kernel,
    mesh=_mesh,
    out_type=jax.ShapeDtypeStruct((NC, NPAD, D), jnp.float32),
    compiler_params=_sc_params,
    scratch_types=[
        pltpu.VMEM((CH, D), jnp.float32),
        pltpu.VMEM((CH, D), jnp.float32),
        pltpu.VMEM((CH, D), jnp.float32),
        pltpu.VMEM((2, CH), jnp.int32),
        pltpu.VMEM((2, CH), jnp.int32),
        pltpu.VMEM((2, CH), jnp.int32),
        pltpu.VMEM((2, CH), jnp.int32),
        pltpu.VMEM_SHARED((NPAD, D), jnp.float32),
        pltpu.SemaphoreType.DMA,
        pltpu.SemaphoreType.DMA,
        pltpu.SemaphoreType.DMA,
        pltpu.SemaphoreType.DMA,
        pltpu.SemaphoreType.DMA,
        pltpu.SemaphoreType.DMA,
        pltpu.SemaphoreType.DMA,
        pltpu.SemaphoreType.DMA,
        pltpu.SemaphoreType.DMA,
        pltpu.SemaphoreType.DMA,
    ],
)
def _edge_kernel(
    h_hbm, eidx_hbm, out_hbm,
    f0, f1, f2, i0, i1, i2, i3, acc,
    gs0, gs1, gs2, ss0, ss1, ss2, is0, is1, is2, is3,
):
    c = lax.axis_index("c")
    s = lax.axis_index("s")
    t = c * NS + s
    F = [f0, f1, f2]
    I = [i0, i1, i2, i3]
    gsem = [gs0, gs1, gs2]
    ssem = [ss0, ss1, ss2]
    isem = [is0, is1, is2, is3]

    def istart(g, sl):
        pltpu.async_copy(eidx_hbm.at[t, g], I[sl], isem[sl])

    def iwait(g, sl):
        pltpu.make_async_copy(eidx_hbm.at[t, g], I[sl], isem[sl]).wait()

    def gstart(b3, sl):
        pltpu.async_copy(h_hbm.at[I[sl].at[0]], F[b3], gsem[b3])

    def gwait(b3, sl):
        pltpu.make_async_copy(h_hbm.at[I[sl].at[0]], F[b3], gsem[b3]).wait()

    def sstart(b3, sl):
        pltpu.async_copy(F[b3], acc.at[I[sl].at[1]], ssem[b3], add=True)

    def swait(b3, sl):
        pltpu.make_async_copy(F[b3], acc.at[I[sl].at[1]], ssem[b3]).wait()

    istart(0, 0)
    istart(1, 1)
    istart(2, 2)
    iwait(0, 0)
    gstart(0, 0)
    iwait(1, 1)
    gstart(1, 1)

    def zbody(i, carry):
        r = i // 8
        j = i % 8
        f2[r, pl.ds(j * 16, 16)] = jnp.zeros((16,), jnp.float32)
        return carry

    lax.fori_loop(0, CH * 8, zbody, 0)

    def zcopy(k, carry):
        pltpu.sync_copy(f2, acc.at[pl.ds(s * RPS + k * CH, CH)])
        return carry

    lax.fori_loop(0, 4, zcopy, 0)

    @pl.when(s < NS - 1)
    def _():
        pltpu.sync_copy(
            f2.at[pl.ds(0, RPS - 4 * CH)],
            acc.at[pl.ds(s * RPS + 4 * CH, RPS - 4 * CH)],
        )

    @pl.when(s == NS - 1)
    def _():
        pltpu.sync_copy(
            f2.at[pl.ds(0, NPAD - 15 * RPS - 4 * CH)],
            acc.at[pl.ds(s * RPS + 4 * CH, NPAD - 15 * RPS - 4 * CH)],
        )

    plsc.subcore_barrier()

    nk = CPT // 12

    def step(k, carry):
        for b in range(12):
            g = 12 * k + b
            b3 = b % 3
            sl = b % 4
            gwait(b3, sl)
            sstart(b3, sl)
            if b == 0:
                @pl.when(k > 0)
                def _():
                    swait(2, 3)
            else:
                swait((b - 1) % 3, (b - 1) % 4)
            if b < 9:
                istart(g + 3, (b + 3) % 4)
            else:
                @pl.when(k < nk - 1)
                def _():
                    istart(g + 3, (b + 3) % 4)
            if b < 10:
                iwait(g + 2, (b + 2) % 4)
                gstart((b + 2) % 3, (b + 2) % 4)
            else:
                @pl.when(k < nk - 1)
                def _():
                    iwait(g + 2, (b + 2) % 4)
                    gstart((b + 2) % 3, (b + 2) % 4)
        return carry

    lax.fori_loop(0, nk, step, 0)
    swait((CPT - 1) % 3, (CPT - 1) % 4)
    plsc.subcore_barrier()

    def ocopy(k, carry):
        pltpu.sync_copy(
            acc.at[pl.ds(s * RPS + k * CH, CH)],
            out_hbm.at[c, pl.ds(s * RPS + k * CH, CH)],
        )
        return carry

    lax.fori_loop(0, 4, ocopy, 0)

    @pl.when(s < NS - 1)
    def _():
        pltpu.sync_copy(
            acc.at[pl.ds(s * RPS + 4 * CH, RPS - 4 * CH)],
            out_hbm.at[c, pl.ds(s * RPS + 4 * CH, RPS - 4 * CH)],
        )

    @pl.when(s == NS - 1)
    def _():
        pltpu.sync_copy(
            acc.at[pl.ds(s * RPS + 4 * CH, NPAD - 15 * RPS - 4 * CH)],
            out_hbm.at[c, pl.ds(s * RPS + 4 * CH, NPAD - 15 * RPS - 4 * CH)],
        )


def _m0_body(x_ref, w_ref, m_ref):
    m_ref[...] = jnp.dot(x_ref[...], w_ref[...], preferred_element_type=jnp.float32)


def _m0(x, w):
    return pl.pallas_call(
        _m0_body,
        out_shape=jax.ShapeDtypeStruct((N, D), jnp.float32),
    )(x, w)


def _scale_body(hists_ref, m_ref, h_ref, dis_ref):
    deg = jnp.sum(hists_ref[:, :N], axis=0) + 1.0
    dis = lax.rsqrt(deg)[:, None]
    dis_ref[...] = dis
    h_ref[...] = m_ref[...] * dis


def _scale(hists, m):
    return pl.pallas_call(
        _scale_body,
        out_shape=(
            jax.ShapeDtypeStruct((N, D), jnp.float32),
            jax.ShapeDtypeStruct((N, 1), jnp.float32),
        ),
    )(hists, m)


def _fuse_body(p_ref, h_ref, dis_ref, b_ref, w_ref, o_ref):
    dis = dis_ref[...]
    a = p_ref[0, :N] + p_ref[1, :N] + h_ref[...]
    x = jnp.maximum(a * dis + b_ref[...], 0.0)
    o_ref[...] = jnp.dot(x * dis, w_ref[...], preferred_element_type=jnp.float32)


def _fuse(p, h, dis, b, w):
    return pl.pallas_call(
        _fuse_body,
        out_shape=jax.ShapeDtypeStruct((N, D), jnp.float32),
    )(p, h, dis, b, w)


def _final_body(p_ref, h_ref, dis_ref, b_ref, o_ref):
    a = p_ref[0, :N] + p_ref[1, :N] + h_ref[...]
    o_ref[...] = a * dis_ref[...] + b_ref[...]


def _final(p, h, dis, b):
    return pl.pallas_call(
        _final_body,
        out_shape=jax.ShapeDtypeStruct((N, D), jnp.float32),
    )(p, h, dis, b)


def kernel(x, edge_index, W0, b0, W1, b1, W2, b2, W3, b3):
    src = edge_index[0].astype(jnp.int32)
    dst = edge_index[1].astype(jnp.int32)
    e = src.shape[0]
    pad_n = EP - e
    ar = jnp.arange(pad_n, dtype=jnp.int32)
    src_p = jnp.concatenate([src, ar % N]).reshape(NT, CPT, CH)
    dst_p = jnp.concatenate([dst, N + ar % (NPAD - N)]).reshape(NT, CPT, CH)
    eidx = jnp.stack([src_p, dst_p], axis=2)
    dst_flat = dst_p.reshape(NT, EPT)

    m = _m0(x, W0)
    hists = _deg_kernel(dst_flat)
    h, dis = _scale(hists, m)
    b_prev = [b0, b1, b2]
    w_next = [W1, W2, W3]
    for i in range(3):
        p = _edge_kernel(h, eidx)
        h = _fuse(p, h, dis, b_prev[i].reshape(1, D), w_next[i])
    p = _edge_kernel(h, eidx)
    return _final(p, h, dis, b3.reshape(1, D))


# speedup vs baseline: 2.1622x; 1.0318x over previous
import functools

import jax
import jax.numpy as jnp
from jax import lax
from jax.experimental import pallas as pl
from jax.experimental.pallas import tpu as pltpu
from jax.experimental.pallas import tpu_sc as plsc

N = 10000
D = 128
NC = 2
NS = 16
NT = NC * NS
CH = 128
CPT = 80
EPT = CPT * CH
EP = NT * EPT
NPAD = 10080
RPS = 632

_mesh = plsc.VectorSubcoreMesh(core_axis_name="c", subcore_axis_name="s")
_sc_params = pltpu.CompilerParams(needs_layout_passes=False)


@functools.partial(
    pl.kernel,
    mesh=_mesh,
    out_type=jax.ShapeDtypeStruct((NT, NPAD), jnp.float32),
    compiler_params=_sc_params,
    scratch_types=[
        pltpu.VMEM((EPT,), jnp.int32),
        pltpu.VMEM((NPAD,), jnp.float32),
    ],
)
def _deg_kernel(dst_hbm, out_hbm, dst_v, hist_v):
    c = lax.axis_index("c")
    s = lax.axis_index("s")
    t = c * NS + s
    pltpu.sync_copy(dst_hbm.at[t], dst_v)

    def zero_body(i, carry):
        hist_v[pl.ds(i * 16, 16)] = jnp.zeros((16,), jnp.float32)
        return carry

    lax.fori_loop(0, NPAD // 16, zero_body, 0)

    ones = jnp.ones((16,), jnp.float32)

    def body(i, carry):
        idx = dst_v[pl.ds(i * 16, 16)]
        plsc.addupdate_scatter(hist_v, [idx], ones)
        return carry

    lax.fori_loop(0, EPT // 16, body, 0)
    pltpu.sync_copy(hist_v, out_hbm.at[t])


@functools.partial(
    pl.kernel,
    mesh=_mesh,
    out_type=jax.ShapeDtypeStruct((NC, NPAD, D), jnp.float32),
    compiler_params=_sc_params,
    scratch_types=[
        pltpu.VMEM((CH, D), jnp.float32),
        pltpu.VMEM((CH, D), jnp.float32),
        pltpu.VMEM((CH, D), jnp.float32),
        pltpu.VMEM((2, CH), jnp.int32),
        pltpu.VMEM((2, CH), jnp.int32),
        pltpu.VMEM((2, CH), jnp.int32),
        pltpu.VMEM((2, CH), jnp.int32),
        pltpu.VMEM_SHARED((NPAD, D), jnp.float32),
        pltpu.SemaphoreType.DMA,
        pltpu.SemaphoreType.DMA,
        pltpu.SemaphoreType.DMA,
        pltpu.SemaphoreType.DMA,
        pltpu.SemaphoreType.DMA,
        pltpu.SemaphoreType.DMA,
        pltpu.SemaphoreType.DMA,
        pltpu.SemaphoreType.DMA,
        pltpu.SemaphoreType.DMA,
        pltpu.SemaphoreType.DMA,
    ],
)
def _edge_kernel(
    h_hbm, eidx_hbm, out_hbm,
    f0, f1, f2, i0, i1, i2, i3, acc,
    gs0, gs1, gs2, ss0, ss1, ss2, is0, is1, is2, is3,
):
    c = lax.axis_index("c")
    s = lax.axis_index("s")
    t = c * NS + s
    F = [f0, f1, f2]
    I = [i0, i1, i2, i3]
    gsem = [gs0, gs1, gs2]
    ssem = [ss0, ss1, ss2]
    isem = [is0, is1, is2, is3]

    def istart(g, sl):
        pltpu.async_copy(eidx_hbm.at[t, g], I[sl], isem[sl])

    def iwait(g, sl):
        pltpu.make_async_copy(eidx_hbm.at[t, g], I[sl], isem[sl]).wait()

    def gstart(b3, sl):
        pltpu.async_copy(h_hbm.at[I[sl].at[0]], F[b3], gsem[b3])

    def gwait(b3, sl):
        pltpu.make_async_copy(h_hbm.at[I[sl].at[0]], F[b3], gsem[b3]).wait()

    def sstart(b3, sl):
        pltpu.async_copy(F[b3], acc.at[I[sl].at[1]], ssem[b3], add=True)

    def swait(b3, sl):
        pltpu.make_async_copy(F[b3], acc.at[I[sl].at[1]], ssem[b3]).wait()

    istart(0, 0)
    istart(1, 1)
    istart(2, 2)
    iwait(0, 0)
    gstart(0, 0)
    iwait(1, 1)
    gstart(1, 1)

    def zbody(i, carry):
        r = i // 8
        j = i % 8
        f2[r, pl.ds(j * 16, 16)] = jnp.zeros((16,), jnp.float32)
        return carry

    lax.fori_loop(0, CH * 8, zbody, 0)

    def zcopy(k, carry):
        pltpu.sync_copy(f2, acc.at[pl.ds(s * RPS + k * CH, CH)])
        return carry

    lax.fori_loop(0, 4, zcopy, 0)

    @pl.when(s < NS - 1)
    def _():
        pltpu.sync_copy(
            f2.at[pl.ds(0, RPS - 4 * CH)],
            acc.at[pl.ds(s * RPS + 4 * CH, RPS - 4 * CH)],
        )

    @pl.when(s == NS - 1)
    def _():
        pltpu.sync_copy(
            f2.at[pl.ds(0, NPAD - 15 * RPS - 4 * CH)],
            acc.at[pl.ds(s * RPS + 4 * CH, NPAD - 15 * RPS - 4 * CH)],
        )

    plsc.subcore_barrier()

    nk = (CPT - 8) // 12

    def step(k, carry):
        for b in range(12):
            g = 12 * k + b
            b3 = b % 3
            sl = b % 4
            gwait(b3, sl)
            sstart(b3, sl)
            if b == 0:
                @pl.when(k > 0)
                def _():
                    swait(2, 3)
            else:
                swait((b - 1) % 3, (b - 1) % 4)
            istart(g + 3, (b + 3) % 4)
            iwait(g + 2, (b + 2) % 4)
            gstart((b + 2) % 3, (b + 2) % 4)
        return carry

    lax.fori_loop(0, nk, step, 0)
    for g in range(CPT - 8, CPT):
        gwait(g % 3, g % 4)
        sstart(g % 3, g % 4)
        swait((g - 1) % 3, (g - 1) % 4)
        if g + 3 < CPT:
            istart(g + 3, (g + 3) % 4)
        if g + 2 < CPT:
            iwait(g + 2, (g + 2) % 4)
            gstart((g + 2) % 3, (g + 2) % 4)
    swait((CPT - 1) % 3, (CPT - 1) % 4)
    plsc.subcore_barrier()

    def ocopy(k, carry):
        pltpu.sync_copy(
            acc.at[pl.ds(s * RPS + k * CH, CH)],
            out_hbm.at[c, pl.ds(s * RPS + k * CH, CH)],
        )
        return carry

    lax.fori_loop(0, 4, ocopy, 0)

    @pl.when(s < NS - 1)
    def _():
        pltpu.sync_copy(
            acc.at[pl.ds(s * RPS + 4 * CH, RPS - 4 * CH)],
            out_hbm.at[c, pl.ds(s * RPS + 4 * CH, RPS - 4 * CH)],
        )

    @pl.when(s == NS - 1)
    def _():
        pltpu.sync_copy(
            acc.at[pl.ds(s * RPS + 4 * CH, NPAD - 15 * RPS - 4 * CH)],
            out_hbm.at[c, pl.ds(s * RPS + 4 * CH, NPAD - 15 * RPS - 4 * CH)],
        )


def _m0_body(x_ref, w_ref, m_ref):
    m_ref[...] = jnp.dot(x_ref[...], w_ref[...], preferred_element_type=jnp.float32)


def _m0(x, w):
    return pl.pallas_call(
        _m0_body,
        out_shape=jax.ShapeDtypeStruct((N, D), jnp.float32),
    )(x, w)


def _scale_body(hists_ref, m_ref, h_ref, dis_ref):
    deg = jnp.sum(hists_ref[:, :N], axis=0) + 1.0
    dis = lax.rsqrt(deg)[:, None]
    dis_ref[...] = dis
    h_ref[...] = m_ref[...] * dis


def _scale(hists, m):
    return pl.pallas_call(
        _scale_body,
        out_shape=(
            jax.ShapeDtypeStruct((N, D), jnp.float32),
            jax.ShapeDtypeStruct((N, 1), jnp.float32),
        ),
    )(hists, m)


def _fuse_body(p_ref, h_ref, dis_ref, b_ref, w_ref, o_ref):
    dis = dis_ref[...]
    a = p_ref[0, :N] + p_ref[1, :N] + h_ref[...]
    x = jnp.maximum(a * dis + b_ref[...], 0.0)
    o_ref[...] = jnp.dot(x * dis, w_ref[...], preferred_element_type=jnp.float32)


def _fuse(p, h, dis, b, w):
    return pl.pallas_call(
        _fuse_body,
        out_shape=jax.ShapeDtypeStruct((N, D), jnp.float32),
    )(p, h, dis, b, w)


def _final_body(p_ref, h_ref, dis_ref, b_ref, o_ref):
    a = p_ref[0, :N] + p_ref[1, :N] + h_ref[...]
    o_ref[...] = a * dis_ref[...] + b_ref[...]


def _final(p, h, dis, b):
    return pl.pallas_call(
        _final_body,
        out_shape=jax.ShapeDtypeStruct((N, D), jnp.float32),
    )(p, h, dis, b)


def kernel(x, edge_index, W0, b0, W1, b1, W2, b2, W3, b3):
    src = edge_index[0].astype(jnp.int32)
    dst = edge_index[1].astype(jnp.int32)
    e = src.shape[0]
    pad_n = EP - e
    ar = jnp.arange(pad_n, dtype=jnp.int32)
    src_p = jnp.concatenate([src, ar % N]).reshape(NT, CPT, CH)
    dst_p = jnp.concatenate([dst, N + ar % (NPAD - N)]).reshape(NT, CPT, CH)
    eidx = jnp.stack([src_p, dst_p], axis=2)
    dst_flat = dst_p.reshape(NT, EPT)

    m = _m0(x, W0)
    hists = _deg_kernel(dst_flat)
    h, dis = _scale(hists, m)
    b_prev = [b0, b1, b2]
    w_next = [W1, W2, W3]
    for i in range(3):
        p = _edge_kernel(h, eidx)
        h = _fuse(p, h, dis, b_prev[i].reshape(1, D), w_next[i])
    p = _edge_kernel(h, eidx)
    return _final(p, h, dis, b3.reshape(1, D))
